# baseline (speedup 1.0000x reference)
import dataclasses
import functools

import jax
import jax.numpy as jnp
from jax import lax
from jax.experimental import pallas as pl
from jax.experimental.pallas import tpu as pltpu
from jax.experimental.pallas import tpu_sc as plsc

_N = 10000
_E = 320000
_D = 128
_R = 64

_NB = 2000
_EB = 2000

_NCHK = 2
_EC = _E // _NCHK
_B = 80
_EPT = _EC // 32
_FB = (_EPT // _B) & ~1
_TAIL = _EPT - _FB * _B
_CH = 16
_NCH = _N // _CH
_CPT = (_NCH + 15) // 16


def _ssp_unit(t):
    return jnp.maximum(t, 0.0) + jnp.log(1.0 + jnp.exp(-jnp.abs(t)))


def _node_mm_body(x_ref, w_ref, o_ref):
    o_ref[...] = jnp.dot(
        x_ref[...].astype(jnp.bfloat16),
        w_ref[...],
        preferred_element_type=jnp.float32,
    )


def _node_mm(x, w1t):
    return pl.pallas_call(
        _node_mm_body,
        out_shape=jax.ShapeDtypeStruct((_N, _D), jnp.float32),
        compiler_params=pltpu.CompilerParams(
            dimension_semantics=("parallel",)
        ),
        grid=(_N // _NB,),
        in_specs=[
            pl.BlockSpec((_NB, _D), lambda i: (i, 0)),
            pl.BlockSpec((_D, _D), lambda i: (0, 0)),
        ],
        out_specs=pl.BlockSpec((_NB, _D), lambda i: (i, 0)),
    )(x, w1t)


def _edge_mlp_body(rbf_ref, wc1_ref, bc1_ref, wc2_ref, bc2_ref, h_ref):
    t = jnp.dot(rbf_ref[...], wc1_ref[...], preferred_element_type=jnp.float32)
    t = _ssp_unit(t + bc1_ref[...])
    h_ref[...] = (
        jnp.dot(
            t.astype(jnp.bfloat16),
            wc2_ref[...],
            preferred_element_type=jnp.float32,
        )
        + bc2_ref[...]
    )


def _edge_mlp(rbf_bf, wc1t, bc1, wc2t, bc2, chunk):
    base = chunk * (_EC // _EB)
    return pl.pallas_call(
        _edge_mlp_body,
        out_shape=jax.ShapeDtypeStruct((_EC, _D), jnp.float32),
        compiler_params=pltpu.CompilerParams(
            dimension_semantics=("parallel",)
        ),
        grid=(_EC // _EB,),
        in_specs=[
            pl.BlockSpec((_EB, _R), lambda i: (i + base, 0)),
            pl.BlockSpec((_R, _D), lambda i: (0, 0)),
            pl.BlockSpec((1, _D), lambda i: (0, 0)),
            pl.BlockSpec((_D, _D), lambda i: (0, 0)),
            pl.BlockSpec((1, _D), lambda i: (0, 0)),
        ],
        out_specs=pl.BlockSpec((_EB, _D), lambda i: (i, 0)),
    )(rbf_bf, wc1t, bc1.reshape(1, _D), wc2t, bc2.reshape(1, _D))


def _final_body(x_ref, p0_ref, p1_ref, p2_ref, p3_ref,
                w2_ref, b2_ref, w3_ref, b3_ref, o_ref):
    cf = (p0_ref[...] + p1_ref[...]) + (p2_ref[...] + p3_ref[...])
    t = jnp.dot(
        cf.astype(jnp.bfloat16), w2_ref[...], preferred_element_type=jnp.float32
    ) + b2_ref[...]
    t = _ssp_unit(t)
    o_ref[...] = (
        x_ref[...]
        + jnp.dot(
            t.astype(jnp.bfloat16), w3_ref[...], preferred_element_type=jnp.float32
        )
        + b3_ref[...]
    )


def _final_mlp(x, parts, w2t, b2, w3t, b3):
    pspec = pl.BlockSpec((_NB, _D), lambda i: (i, 0))
    return pl.pallas_call(
        _final_body,
        out_shape=jax.ShapeDtypeStruct((_N, _D), jnp.float32),
        compiler_params=pltpu.CompilerParams(
            dimension_semantics=("parallel",)
        ),
        grid=(_N // _NB,),
        in_specs=[
            pspec, pspec, pspec, pspec, pspec,
            pl.BlockSpec((_D, _D), lambda i: (0, 0)),
            pl.BlockSpec((1, _D), lambda i: (0, 0)),
            pl.BlockSpec((_D, _D), lambda i: (0, 0)),
            pl.BlockSpec((1, _D), lambda i: (0, 0)),
        ],
        out_specs=pl.BlockSpec((_NB, _D), lambda i: (i, 0)),
    )(x, *parts, w2t, b2.reshape(1, _D), w3t, b3.reshape(1, _D))


def _sc_msgpass(new_node, h, src, dst, chunk):
    mesh = plsc.VectorSubcoreMesh(core_axis_name="c", subcore_axis_name="s")
    cp = pltpu.CompilerParams()
    if "needs_layout_passes" in pltpu.CompilerParams.__dataclass_fields__:
        cp = dataclasses.replace(cp, needs_layout_passes=False)

    @functools.partial(
        pl.kernel,
        out_type=jax.ShapeDtypeStruct((2 * _N, _D), jnp.float32),
        mesh=mesh,
        compiler_params=cp,
        scratch_types=[
            pltpu.VMEM((_EPT,), jnp.int32),
            pltpu.VMEM((_B,), jnp.int32),
            pltpu.VMEM((_B,), jnp.int32),
            pltpu.VMEM((_TAIL,), jnp.int32),
            pltpu.VMEM((_B, _D), jnp.float32),
            pltpu.VMEM((_B, _D), jnp.float32),
            pltpu.VMEM((_B, _D), jnp.float32),
            pltpu.VMEM((_B, _D), jnp.float32),
            pltpu.VMEM((_CH, _D), jnp.float32),
            pltpu.VMEM_SHARED((_N, _D), jnp.float32),
            pltpu.SemaphoreType.DMA,
            pltpu.SemaphoreType.DMA,
            pltpu.SemaphoreType.DMA,
            pltpu.SemaphoreType.DMA,
            pltpu.SemaphoreType.DMA,
            pltpu.SemaphoreType.DMA,
        ],
    )
    def k(nn_hbm, h_hbm, src_hbm, dst_hbm, out_hbm,
          src_all, dst0, dst1, dst_t, rows0, rows1, h0, h1, zbuf, acc,
          sg0, sg1, sh0, sh1, sd0, sd1):
        cid = lax.axis_index("c")
        sid = lax.axis_index("s")
        w = sid * 2 + cid
        hb0 = w * _EPT
        eb = chunk * _EC + hb0

        bufs = ((dst0, rows0, h0, sg0, sh0, sd0),
                (dst1, rows1, h1, sg1, sh1, sd1))

        def issue(kk, dstb, rowsb, hb, sg, sh, sd):
            base = eb + kk * _B
            pltpu.async_copy(dst_hbm.at[pl.ds(base, _B)], dstb, sd)
            pltpu.async_copy(
                nn_hbm.at[src_all.at[pl.ds(kk * _B, _B)]], rowsb, sg)
            pltpu.async_copy(h_hbm.at[pl.ds(hb0 + kk * _B, _B)], hb, sh)

        def process(kk, dstb, rowsb, hb, sg, sh, sd):
            base = eb + kk * _B
            pltpu.make_async_copy(dst_hbm.at[pl.ds(base, _B)], dstb, sd).wait()
            pltpu.make_async_copy(
                nn_hbm.at[src_all.at[pl.ds(kk * _B, _B)]], rowsb, sg).wait()
            pltpu.make_async_copy(
                h_hbm.at[pl.ds(hb0 + kk * _B, _B)], hb, sh).wait()

            @plsc.parallel_loop(0, _B)
            def _(e):
                for j in range(8):
                    sl = pl.ds(j * 16, 16)
                    rowsb[e, sl] = rowsb[e, sl] * hb[e, sl]

            pltpu.sync_copy(rowsb, acc.at[dstb], add=True)

        pltpu.sync_copy(src_hbm.at[pl.ds(eb, _EPT)], src_all)
        issue(0, *bufs[0])
        issue(1, *bufs[1])

        @pl.loop(0, _CH)
        def _(i):
            for j in range(8):
                zbuf[i, pl.ds(j * 16, 16)] = jnp.zeros((16,), jnp.float32)

        @pl.loop(0, _CPT)
        def _(i):
            c = sid + 16 * i

            @pl.when(c < _NCH)
            def _():
                pltpu.sync_copy(zbuf, acc.at[pl.ds(c * _CH, _CH)])

        plsc.subcore_barrier()

        @pl.loop(0, _FB // 2)
        def _(p):
            for b in range(2):
                kk = 2 * p + b
                process(kk, *bufs[b])

                @pl.when(kk + 2 < _FB)
                def _():
                    issue(kk + 2, *bufs[b])

        pltpu.sync_copy(dst_hbm.at[pl.ds(eb + _FB * _B, _TAIL)], dst_t)
        pltpu.async_copy(
            nn_hbm.at[src_all.at[pl.ds(_FB * _B, _TAIL)]],
            rows0.at[pl.ds(0, _TAIL)], sg0).wait()
        pltpu.async_copy(
            h_hbm.at[pl.ds(hb0 + _FB * _B, _TAIL)],
            h0.at[pl.ds(0, _TAIL)], sh0).wait()

        @plsc.parallel_loop(0, _TAIL)
        def _(e):
            for j in range(8):
                sl = pl.ds(j * 16, 16)
                rows0[e, sl] = rows0[e, sl] * h0[e, sl]

        pltpu.sync_copy(rows0.at[pl.ds(0, _TAIL)], acc.at[dst_t], add=True)

        plsc.subcore_barrier()
        @pl.loop(0, _CPT)
        def _(i):
            c = sid + 16 * i

            @pl.when(c < _NCH)
            def _():
                pltpu.sync_copy(
                    acc.at[pl.ds(c * _CH, _CH)],
                    out_hbm.at[pl.ds(cid * _N + c * _CH, _CH)],
                )

    return k(new_node, h, src, dst)


def kernel(x, edge_index, rbf, W1, Wc1, bc1, Wc2, bc2, W2, b2, W3, b3):
    src = edge_index[0]
    dst = edge_index[1]

    w1t = W1.T.astype(jnp.bfloat16)
    wc1t = (0.5 * Wc1.T).astype(jnp.bfloat16)
    bc1h = 0.5 * bc1
    wc2t = (2.0 * Wc2.T).astype(jnp.bfloat16)
    w2t = (0.5 * W2.T).astype(jnp.bfloat16)
    b2h = 0.5 * b2
    w3t = (2.0 * W3.T).astype(jnp.bfloat16)
    rbf_bf = rbf.astype(jnp.bfloat16)

    new_node = _node_mm(x, w1t)
    parts = []
    for c in range(_NCHK):
        h_c = _edge_mlp(rbf_bf, wc1t, bc1h, wc2t, bc2, c)
        p = _sc_msgpass(new_node, h_c, src, dst, c)
        parts.extend([p[:_N], p[_N:]])
    return _final_mlp(x, parts, w2t, b2h, w3t, b3)


# speedup vs baseline: 1.0497x; 1.0497x over previous
import dataclasses
import functools

import jax
import jax.numpy as jnp
from jax import lax
from jax.experimental import pallas as pl
from jax.experimental.pallas import tpu as pltpu
from jax.experimental.pallas import tpu_sc as plsc

_N = 10000
_E = 320000
_D = 128
_R = 64

_NB = 2000
_EB = 2000

_NCHK = 2
_EC = _E // _NCHK
_B = 64
_EPT = _EC // 32
_FB = (_EPT // _B) & ~1
_TAIL = _EPT - _FB * _B
_CH = 40
_NCH = _N // _CH
_CPT = (_NCH + 15) // 16


def _ssp_unit(t):
    return jnp.maximum(t, 0.0) + jnp.log(1.0 + jnp.exp(-jnp.abs(t)))


def _node_mm_body(x_ref, w_ref, o_ref):
    o_ref[...] = jnp.dot(
        x_ref[...].astype(jnp.bfloat16),
        w_ref[...],
        preferred_element_type=jnp.float32,
    )


def _node_mm(x, w1t):
    return pl.pallas_call(
        _node_mm_body,
        out_shape=jax.ShapeDtypeStruct((_N, _D), jnp.float32),
        compiler_params=pltpu.CompilerParams(
            dimension_semantics=("parallel",)
        ),
        grid=(_N // _NB,),
        in_specs=[
            pl.BlockSpec((_NB, _D), lambda i: (i, 0)),
            pl.BlockSpec((_D, _D), lambda i: (0, 0)),
        ],
        out_specs=pl.BlockSpec((_NB, _D), lambda i: (i, 0)),
    )(x, w1t)


def _edge_mlp_body(rbf_ref, wc1_ref, bc1_ref, wc2_ref, bc2_ref, h_ref):
    t = jnp.dot(rbf_ref[...], wc1_ref[...], preferred_element_type=jnp.float32)
    t = _ssp_unit(t + bc1_ref[...])
    h_ref[...] = (
        jnp.dot(
            t.astype(jnp.bfloat16),
            wc2_ref[...],
            preferred_element_type=jnp.float32,
        )
        + bc2_ref[...]
    )


def _edge_mlp(rbf_bf, wc1t, bc1, wc2t, bc2, chunk):
    base = chunk * (_EC // _EB)
    return pl.pallas_call(
        _edge_mlp_body,
        out_shape=jax.ShapeDtypeStruct((_EC, _D), jnp.float32),
        compiler_params=pltpu.CompilerParams(
            dimension_semantics=("parallel",)
        ),
        grid=(_EC // _EB,),
        in_specs=[
            pl.BlockSpec((_EB, _R), lambda i: (i + base, 0)),
            pl.BlockSpec((_R, _D), lambda i: (0, 0)),
            pl.BlockSpec((1, _D), lambda i: (0, 0)),
            pl.BlockSpec((_D, _D), lambda i: (0, 0)),
            pl.BlockSpec((1, _D), lambda i: (0, 0)),
        ],
        out_specs=pl.BlockSpec((_EB, _D), lambda i: (i, 0)),
    )(rbf_bf, wc1t, bc1.reshape(1, _D), wc2t, bc2.reshape(1, _D))


def _final_body(x_ref, p0_ref, p1_ref, p2_ref, p3_ref,
                w2_ref, b2_ref, w3_ref, b3_ref, o_ref):
    cf = (p0_ref[...] + p1_ref[...]) + (p2_ref[...] + p3_ref[...])
    t = jnp.dot(
        cf.astype(jnp.bfloat16), w2_ref[...], preferred_element_type=jnp.float32
    ) + b2_ref[...]
    t = _ssp_unit(t)
    o_ref[...] = (
        x_ref[...]
        + jnp.dot(
            t.astype(jnp.bfloat16), w3_ref[...], preferred_element_type=jnp.float32
        )
        + b3_ref[...]
    )


def _final_mlp(x, parts, w2t, b2, w3t, b3):
    pspec = pl.BlockSpec((_NB, _D), lambda i: (i, 0))
    return pl.pallas_call(
        _final_body,
        out_shape=jax.ShapeDtypeStruct((_N, _D), jnp.float32),
        compiler_params=pltpu.CompilerParams(
            dimension_semantics=("parallel",)
        ),
        grid=(_N // _NB,),
        in_specs=[
            pspec, pspec, pspec, pspec, pspec,
            pl.BlockSpec((_D, _D), lambda i: (0, 0)),
            pl.BlockSpec((1, _D), lambda i: (0, 0)),
            pl.BlockSpec((_D, _D), lambda i: (0, 0)),
            pl.BlockSpec((1, _D), lambda i: (0, 0)),
        ],
        out_specs=pl.BlockSpec((_NB, _D), lambda i: (i, 0)),
    )(x, *parts, w2t, b2.reshape(1, _D), w3t, b3.reshape(1, _D))


def _sc_msgpass(new_node, h, src, dst, chunk):
    mesh = plsc.VectorSubcoreMesh(core_axis_name="c", subcore_axis_name="s")
    cp = pltpu.CompilerParams()
    if "needs_layout_passes" in pltpu.CompilerParams.__dataclass_fields__:
        cp = dataclasses.replace(cp, needs_layout_passes=False)

    @functools.partial(
        pl.kernel,
        out_type=jax.ShapeDtypeStruct((2 * _N, _D), jnp.float32),
        mesh=mesh,
        compiler_params=cp,
        scratch_types=[
            pltpu.VMEM((_EPT,), jnp.int32),
            pltpu.VMEM((_B,), jnp.int32),
            pltpu.VMEM((_B,), jnp.int32),
            pltpu.VMEM((_TAIL,), jnp.int32),
            pltpu.VMEM((_B, _D), jnp.float32),
            pltpu.VMEM((_B, _D), jnp.float32),
            pltpu.VMEM((_B, _D), jnp.float32),
            pltpu.VMEM((_B, _D), jnp.float32),
            pltpu.VMEM((_CH, _D), jnp.float32),
            pltpu.VMEM_SHARED((_N, _D), jnp.float32),
            pltpu.SemaphoreType.DMA,
            pltpu.SemaphoreType.DMA,
            pltpu.SemaphoreType.DMA,
            pltpu.SemaphoreType.DMA,
            pltpu.SemaphoreType.DMA,
            pltpu.SemaphoreType.DMA,
        ],
    )
    def k(nn_hbm, h_hbm, src_hbm, dst_hbm, out_hbm,
          src_all, dst0, dst1, dst_t, rows0, rows1, h0, h1, zbuf, acc,
          sg0, sg1, sh0, sh1, sd0, sd1):
        cid = lax.axis_index("c")
        sid = lax.axis_index("s")
        w = sid * 2 + cid
        hb0 = w * _EPT
        eb = chunk * _EC + hb0

        bufs = ((dst0, rows0, h0, sg0, sh0, sd0),
                (dst1, rows1, h1, sg1, sh1, sd1))

        def issue(kk, dstb, rowsb, hb, sg, sh, sd):
            base = eb + kk * _B
            pltpu.async_copy(dst_hbm.at[pl.ds(base, _B)], dstb, sd)
            pltpu.async_copy(
                nn_hbm.at[src_all.at[pl.ds(kk * _B, _B)]], rowsb, sg)
            pltpu.async_copy(h_hbm.at[pl.ds(hb0 + kk * _B, _B)], hb, sh)

        def process(kk, dstb, rowsb, hb, sg, sh, sd):
            base = eb + kk * _B
            pltpu.make_async_copy(dst_hbm.at[pl.ds(base, _B)], dstb, sd).wait()
            pltpu.make_async_copy(
                nn_hbm.at[src_all.at[pl.ds(kk * _B, _B)]], rowsb, sg).wait()
            pltpu.make_async_copy(
                h_hbm.at[pl.ds(hb0 + kk * _B, _B)], hb, sh).wait()

            @plsc.parallel_loop(0, _B)
            def _(e):
                for j in range(8):
                    sl = pl.ds(j * 16, 16)
                    rowsb[e, sl] = rowsb[e, sl] * hb[e, sl]

            pltpu.sync_copy(rowsb, acc.at[dstb], add=True)

        pltpu.sync_copy(src_hbm.at[pl.ds(eb, _EPT)], src_all)
        issue(0, *bufs[0])
        issue(1, *bufs[1])

        @pl.loop(0, _CH)
        def _(i):
            for j in range(8):
                zbuf[i, pl.ds(j * 16, 16)] = jnp.zeros((16,), jnp.float32)

        @pl.loop(0, _CPT)
        def _(i):
            c = sid + 16 * i

            @pl.when(c < _NCH)
            def _():
                pltpu.sync_copy(zbuf, acc.at[pl.ds(c * _CH, _CH)])

        plsc.subcore_barrier()

        @pl.loop(0, _FB // 2)
        def _(p):
            for b in range(2):
                kk = 2 * p + b
                process(kk, *bufs[b])

                @pl.when(kk + 2 < _FB)
                def _():
                    issue(kk + 2, *bufs[b])

        pltpu.sync_copy(dst_hbm.at[pl.ds(eb + _FB * _B, _TAIL)], dst_t)
        pltpu.async_copy(
            nn_hbm.at[src_all.at[pl.ds(_FB * _B, _TAIL)]],
            rows0.at[pl.ds(0, _TAIL)], sg0).wait()
        pltpu.async_copy(
            h_hbm.at[pl.ds(hb0 + _FB * _B, _TAIL)],
            h0.at[pl.ds(0, _TAIL)], sh0).wait()

        @plsc.parallel_loop(0, _TAIL)
        def _(e):
            for j in range(8):
                sl = pl.ds(j * 16, 16)
                rows0[e, sl] = rows0[e, sl] * h0[e, sl]

        pltpu.sync_copy(rows0.at[pl.ds(0, _TAIL)], acc.at[dst_t], add=True)

        plsc.subcore_barrier()
        @pl.loop(0, _CPT)
        def _(i):
            c = sid + 16 * i

            @pl.when(c < _NCH)
            def _():
                pltpu.sync_copy(
                    acc.at[pl.ds(c * _CH, _CH)],
                    out_hbm.at[pl.ds(cid * _N + c * _CH, _CH)],
                )

    return k(new_node, h, src, dst)


def kernel(x, edge_index, rbf, W1, Wc1, bc1, Wc2, bc2, W2, b2, W3, b3):
    src = edge_index[0]
    dst = edge_index[1]

    w1t = W1.T.astype(jnp.bfloat16)
    wc1t = (0.5 * Wc1.T).astype(jnp.bfloat16)
    bc1h = 0.5 * bc1
    wc2t = (2.0 * Wc2.T).astype(jnp.bfloat16)
    w2t = (0.5 * W2.T).astype(jnp.bfloat16)
    b2h = 0.5 * b2
    w3t = (2.0 * W3.T).astype(jnp.bfloat16)
    rbf_bf = rbf.astype(jnp.bfloat16)

    new_node = _node_mm(x, w1t)
    parts = []
    for c in range(_NCHK):
        h_c = _edge_mlp(rbf_bf, wc1t, bc1h, wc2t, bc2, c)
        p = _sc_msgpass(new_node, h_c, src, dst, c)
        parts.extend([p[:_N], p[_N:]])
    return _final_mlp(x, parts, w2t, b2h, w3t, b3)
